# baseline (speedup 1.0000x reference)
import jax
import jax.numpy as jnp
from jax.experimental import pallas as pl
from jax.experimental.pallas import tpu as pltpu

_BLOCK_ROWS = 5000
_SUB = 1000
_NSUB = _BLOCK_ROWS // _SUB


def _fused_linear_relu(h_ref, w_ref, b_ref, o_hbm, obuf, osem):
    i = pl.program_id(0)
    nsteps = pl.num_programs(0)
    w = w_ref[...].astype(jnp.bfloat16)
    bias = b_ref[...]

    def out_copy(step, s):
        return pltpu.make_async_copy(
            obuf.at[s],
            o_hbm.at[pl.ds(step * _BLOCK_ROWS + s * _SUB, _SUB), :],
            osem.at[s])

    for s in range(_NSUB):
        x = h_ref[pl.ds(s * _SUB, _SUB), :].astype(jnp.bfloat16)
        acc = jax.lax.dot_general(
            x, w, (((1,), (1,)), ((), ())),
            preferred_element_type=jnp.float32)
        y = jnp.maximum(acc + bias, 0.0)

        @pl.when(i > 0)
        def _drain_prev():
            out_copy(i - 1, s).wait()

        obuf[s] = y
        out_copy(i, s).start()

    @pl.when(i == nsteps - 1)
    def _drain_all():
        for s in range(_NSUB):
            out_copy(i, s).wait()


def kernel(h, edge_index, W, b):
    del edge_index
    n, d_in = h.shape
    d_out = W.shape[0]
    b2 = b.reshape(1, d_out)
    return pl.pallas_call(
        _fused_linear_relu,
        grid=(n // _BLOCK_ROWS,),
        in_specs=[
            pl.BlockSpec((_BLOCK_ROWS, d_in), lambda i: (i, 0)),
            pl.BlockSpec((d_out, d_in), lambda i: (0, 0)),
            pl.BlockSpec((1, d_out), lambda i: (0, 0)),
        ],
        out_specs=pl.BlockSpec(memory_space=pl.ANY),
        out_shape=jax.ShapeDtypeStruct((n, d_out), jnp.float32),
        scratch_shapes=[
            pltpu.VMEM((_NSUB, _SUB, d_out), jnp.float32),
            pltpu.SemaphoreType.DMA((_NSUB,)),
        ],
    )(h, W, b2)
